# baseline (speedup 1.0000x reference)
import functools

import jax
import jax.numpy as jnp
from jax import lax
from jax.experimental import pallas as pl
from jax.experimental.pallas import tpu as pltpu
from jax.experimental.pallas import tpu_sc as plsc

N_CELLS = 4096
N_GENES = 16384
N_F = 128

_N_SUB = 16
_SLICE = N_GENES // _N_SUB
_ROWS_PER_W = N_CELLS // 32
_STAGE = 144

_I32_MAX = 2**31 - 1


def _lane_scalar(vec, lane, i16):
    return jnp.min(jnp.where(i16 == lane, vec, _I32_MAX))


def _sc_filter_body(
    x_hbm,
    var_hbm,
    o_hbm,
    vf_hbm,
    vv,
    st,
    lc,
    pos_full,
    ivals,
    vvals,
    idx_v,
    buf_o,
    buf_w,
    sh_cnt,
    sh_idx,
    sh_vf,
):
    core = lax.axis_index("c")
    sub = lax.axis_index("s")
    i16 = lax.iota(jnp.int32, 16)
    zeros16 = jnp.zeros((16,), jnp.int32)

    gbase = pl.multiple_of(sub * _SLICE, 8)
    pltpu.sync_copy(var_hbm.at[pl.ds(gbase, _SLICE)], vv)

    @pl.loop(0, _SLICE // 16, init_carry=zeros16)
    def count_loop(c, cnt):
        off = pl.multiple_of(c * 16, 8)
        vc = vv[pl.ds(off, 16)]
        m = (vc >= 0) & (vc < N_F)
        return cnt + plsc.all_reduce_population_count(m)

    st[pl.ds(0, 16)] = count_loop
    pltpu.sync_copy(st.at[pl.ds(0, 16)], sh_cnt.at[sub])

    @pl.when(sub == 0)
    def _init_shared():
        var0 = _lane_scalar(vv[pl.ds(0, 16)], 0, i16)
        for c in range(_STAGE // 16):
            st[pl.ds(c * 16, 16)] = zeros16
        pltpu.sync_copy(st, sh_idx)
        v0v = jnp.full((16,), var0, jnp.int32)
        for c in range(_STAGE // 16):
            st[pl.ds(c * 16, 16)] = v0v
        pltpu.sync_copy(st, sh_vf)

    plsc.subcore_barrier()

    pltpu.sync_copy(sh_cnt, lc)
    counts = plsc.load_gather(lc, [i16, zeros16])
    pref = jnp.sum(jnp.where(i16 < sub, counts, 0))

    for c in range(_STAGE // 16):
        pos_full[pl.ds(c * 16, 16)] = jnp.full((16,), N_F, jnp.int32)

    @pl.loop(0, _SLICE // 16, init_carry=zeros16)
    def scan_loop(c, run):
        off = pl.multiple_of(c * 16, 8)
        vc = vv[pl.ds(off, 16)]
        m = (vc >= 0) & (vc < N_F)
        cs = plsc.cumsum(m.astype(jnp.int32))
        ordv = run + cs - 1
        gpos = ordv + pref
        tgt = jnp.where(gpos < N_F, gpos, N_F)
        ordc = jnp.minimum(ordv, _STAGE - 1)
        plsc.store_scatter(pos_full, [ordc], tgt, mask=m)
        plsc.store_scatter(ivals, [ordc], gbase + c * 16 + i16, mask=m)
        plsc.store_scatter(vvals, [ordc], vc, mask=m)
        return run + plsc.all_reduce_population_count(m)

    pltpu.sync_copy(ivals, sh_idx.at[pos_full])
    pltpu.sync_copy(vvals, sh_vf.at[pos_full])
    plsc.subcore_barrier()

    pltpu.sync_copy(sh_idx.at[pl.ds(0, N_F)], idx_v)

    @pl.when((sub == 0) & (core == 0))
    def _write_vf():
        pltpu.sync_copy(sh_vf.at[pl.ds(0, N_F)], vf_hbm)

    w = sub * 2 + core
    row0 = w * _ROWS_PER_W

    idx0 = _lane_scalar(idx_v[pl.ds(0, 16)], 0, i16)
    acc = jnp.ones((16,), dtype=jnp.bool_)
    for c in range(N_F // 16):
        vc = idx_v[pl.ds(c * 16, 16)]
        acc = acc & (vc == idx0 + c * 16 + i16)
    fast = jnp.all(acc) & (lax.rem(idx0, 128) == 0)

    @pl.when(fast)
    def _fast():
        src0 = pl.multiple_of(idx0, 128)
        pltpu.sync_copy(
            x_hbm.at[pl.ds(row0, _ROWS_PER_W), pl.ds(src0, N_F)], buf_o
        )

    @pl.when(jnp.logical_not(fast))
    def _slow():
        @pl.loop(0, N_F)
        def _(k):
            cbase = pl.multiple_of((k // 16) * 16, 8)
            chunk = idx_v[pl.ds(cbase, 16)]
            oj = _lane_scalar(chunk, lax.rem(k, 16), i16)
            a = pl.multiple_of((oj // 128) * 128, 128)
            r = oj - a
            pltpu.sync_copy(
                x_hbm.at[pl.ds(row0, _ROWS_PER_W), pl.ds(a, 128)], buf_w
            )

            @pl.loop(0, _ROWS_PER_W // 16)
            def _(i):
                rows = i * 16 + i16
                vals = plsc.load_gather(buf_w, [rows, jnp.full((16,), r, jnp.int32)])
                plsc.store_scatter(buf_o, [rows, jnp.full((16,), k, jnp.int32)], vals)

    pltpu.sync_copy(buf_o, o_hbm.at[pl.ds(row0, _ROWS_PER_W)])


def _sc_filter(x_ng, var32):
    mesh = plsc.VectorSubcoreMesh(core_axis_name="c", subcore_axis_name="s")
    return pl.kernel(
        _sc_filter_body,
        out_type=(
            jax.ShapeDtypeStruct((N_CELLS, N_F), x_ng.dtype),
            jax.ShapeDtypeStruct((N_F,), jnp.int32),
        ),
        mesh=mesh,
        compiler_params=pltpu.CompilerParams(needs_layout_passes=False),
        scratch_types=[
            pltpu.VMEM((_SLICE,), jnp.int32),
            pltpu.VMEM((_STAGE,), jnp.int32),
            pltpu.VMEM((_N_SUB, 16), jnp.int32),
            pltpu.VMEM((_STAGE,), jnp.int32),
            pltpu.VMEM((_STAGE,), jnp.int32),
            pltpu.VMEM((_STAGE,), jnp.int32),
            pltpu.VMEM((N_F,), jnp.int32),
            pltpu.VMEM((_ROWS_PER_W, N_F), x_ng.dtype),
            pltpu.VMEM((_ROWS_PER_W, 128), x_ng.dtype),
            pltpu.VMEM_SHARED((_N_SUB, 16), jnp.int32),
            pltpu.VMEM_SHARED((_STAGE,), jnp.int32),
            pltpu.VMEM_SHARED((_STAGE,), jnp.int32),
        ],
    )(x_ng, var32)


def kernel(x_ng, var_names_g):
    var32 = var_names_g.astype(jnp.int32)
    x_filtered, vf = _sc_filter(x_ng, var32)
    return (x_filtered, vf.astype(var_names_g.dtype))
